# Initial kernel scaffold; baseline (speedup 1.0000x reference)
import jax
import jax.numpy as jnp
from jax.experimental import pallas as pl


def kernel(x, edge_index, weight):
    raise NotImplementedError("write your pallas kernel here")


# speedup vs baseline: 3.5167x; 3.5167x over previous
import functools

import jax
import jax.numpy as jnp
from jax import lax
from jax.experimental import pallas as pl
from jax.experimental.pallas import tpu as pltpu
from jax.experimental.pallas import tpu_sc as plsc

N = 10000
D = 128
NC = 2
NS = 16
NW = NC * NS
CHUNK = 128
NCHUNKS = 80
GB = 8
EDGES_PER_TILE = NCHUNKS * CHUNK
E_PAD = NW * EDGES_PER_TILE
ROWS_PER_TILE = 632
NP = NS * ROWS_PER_TILE


def _sc_aggregate(x, src_r, dst_r, dst_f, zsum):
    mesh = plsc.VectorSubcoreMesh(core_axis_name="c", subcore_axis_name="s")

    @functools.partial(
        pl.kernel,
        mesh=mesh,
        compiler_params=pltpu.CompilerParams(needs_layout_passes=False),
        out_type=(
            jax.ShapeDtypeStruct((NC, NP, D), jnp.float32),
            jax.ShapeDtypeStruct((NW, NP), jnp.float32),
        ),
        scratch_types=[
            pltpu.VMEM_SHARED((NP, D), jnp.float32),
            pltpu.VMEM((GB, CHUNK), jnp.int32),
            pltpu.VMEM((GB, CHUNK), jnp.int32),
            pltpu.VMEM((GB * CHUNK,), jnp.int32),
            pltpu.VMEM((CHUNK, D), jnp.float32),
            pltpu.VMEM((NP,), jnp.float32),
            pltpu.SemaphoreType.DMA,
        ],
    )
    def k(x_hbm, src_hbm, dst_hbm, dstf_hbm, zsum_hbm, sum_out, cnt_out,
          accum, src_v, dst_v, dst1d, rows_v, cnt_v, sem):
        c = lax.axis_index("c")
        s = lax.axis_index("s")
        wid = c * NS + s
        rows_slice = pl.ds(s * ROWS_PER_TILE, ROWS_PER_TILE)
        pltpu.sync_copy(zsum_hbm.at[rows_slice], accum.at[rows_slice])
        zero16 = jnp.zeros((16,), jnp.float32)
        ones16 = jnp.ones((16,), jnp.float32)

        def zbody(i, carry):
            cnt_v[pl.ds(i * 16, 16)] = zero16
            return carry

        lax.fori_loop(0, NP // 16, zbody, 0)
        plsc.subcore_barrier()

        def outer(g, carry):
            goff = pl.multiple_of(g * GB, GB)
            goff2 = pl.multiple_of(g * GB * CHUNK, GB * CHUNK)
            pltpu.sync_copy(src_hbm.at[wid].at[pl.ds(goff, GB)], src_v)
            pltpu.sync_copy(dst_hbm.at[wid].at[pl.ds(goff, GB)], dst_v)
            pltpu.sync_copy(dstf_hbm.at[wid].at[pl.ds(goff2, GB * CHUNK)], dst1d)

            def body(j, carry2):
                pltpu.async_copy(x_hbm.at[src_v.at[j]], rows_v, sem).wait()
                pltpu.sync_copy(rows_v, accum.at[dst_v.at[j]], add=True)
                return carry2

            lax.fori_loop(0, GB, body, 0)

            def cbody(m, carry2):
                vals = dst1d[pl.ds(m * 16, 16)]
                plsc.addupdate_scatter(cnt_v, [vals], ones16)
                return carry2

            lax.fori_loop(0, GB * CHUNK // 16, cbody, 0)
            return carry

        lax.fori_loop(0, NCHUNKS // GB, outer, 0)
        plsc.subcore_barrier()
        pltpu.sync_copy(accum.at[rows_slice], sum_out.at[c].at[rows_slice])
        pltpu.sync_copy(cnt_v, cnt_out.at[wid])

    return k(x, src_r, dst_r, dst_f, zsum)


def _tc_combine(x, w1, w2, sp, cp, n_dummy):
    R = 1000

    def body(x_ref, w1_ref, w2_ref, sp_ref, cp_ref, o_ref):
        j = pl.program_id(0)
        ssum = sp_ref[0] + sp_ref[1]
        cnt = jnp.sum(cp_ref[...], axis=1, keepdims=True)
        rows = lax.broadcasted_iota(jnp.int32, (R, 1), 0)
        corr = jnp.where(jnp.logical_and(rows == 0, j == 0),
                         jnp.float32(n_dummy), jnp.float32(0.0))
        xb = x_ref[...]
        ssum = ssum - corr * xb
        cnt = jnp.maximum(cnt - corr, 1.0)
        mean = ssum / cnt
        o_ref[...] = (
            jnp.dot(xb, w1_ref[...], preferred_element_type=jnp.float32)
            + jnp.dot(mean, w2_ref[...], preferred_element_type=jnp.float32))

    return pl.pallas_call(
        body,
        grid=(N // R,),
        in_specs=[
            pl.BlockSpec((R, D), lambda j: (j, 0)),
            pl.BlockSpec((D, D), lambda j: (0, 0)),
            pl.BlockSpec((D, D), lambda j: (0, 0)),
            pl.BlockSpec((NC, R, D), lambda j: (0, j, 0)),
            pl.BlockSpec((R, NW), lambda j: (j, 0)),
        ],
        out_specs=pl.BlockSpec((R, D), lambda j: (j, 0)),
        out_shape=jax.ShapeDtypeStruct((N, D), jnp.float32),
    )(x, w1, w2, sp, cp)


def kernel(x, edge_index, weight):
    src = edge_index[0]
    dst = edge_index[1]
    e = src.shape[0]
    pad = E_PAD - e
    src_p = jnp.concatenate([src, jnp.zeros((pad,), jnp.int32)])
    dst_p = jnp.concatenate([dst, jnp.zeros((pad,), jnp.int32)])
    src_r = src_p.reshape(NW, NCHUNKS, CHUNK)
    dst_r = dst_p.reshape(NW, NCHUNKS, CHUNK)
    dst_f = dst_p.reshape(NW, EDGES_PER_TILE)
    zsum = jnp.zeros((NP, D), jnp.float32)
    sp, cnt_parts = _sc_aggregate(x, src_r, dst_r, dst_f, zsum)
    sp = sp[:, :N]
    cp = cnt_parts[:, :N].T
    w1 = weight[:D]
    w2 = weight[D:]
    return _tc_combine(x, w1, w2, sp, cp, float(pad))


# speedup vs baseline: 3.8968x; 1.1081x over previous
import functools

import jax
import jax.numpy as jnp
from jax import lax
from jax.experimental import pallas as pl
from jax.experimental.pallas import tpu as pltpu
from jax.experimental.pallas import tpu_sc as plsc

N = 10000
D = 128
NC = 2
NS = 16
NW = NC * NS
CHUNK = 128
NCHUNKS = 80
GB = 8
EDGES_PER_TILE = NCHUNKS * CHUNK
E_PAD = NW * EDGES_PER_TILE
ROWS_PER_TILE = 632
NP = NS * ROWS_PER_TILE


def _sc_aggregate(x, src_r, dst_r, dst_f, zsum):
    mesh = plsc.VectorSubcoreMesh(core_axis_name="c", subcore_axis_name="s")

    @functools.partial(
        pl.kernel,
        mesh=mesh,
        compiler_params=pltpu.CompilerParams(needs_layout_passes=False),
        out_type=(
            jax.ShapeDtypeStruct((NC, NP, D), jnp.float32),
            jax.ShapeDtypeStruct((NW, NP), jnp.float32),
        ),
        scratch_types=[
            pltpu.VMEM_SHARED((NP, D), jnp.float32),
            pltpu.VMEM((GB, CHUNK), jnp.int32),
            pltpu.VMEM((GB, CHUNK), jnp.int32),
            pltpu.VMEM((GB * CHUNK,), jnp.int32),
            pltpu.VMEM((CHUNK, D), jnp.float32),
            pltpu.VMEM((CHUNK, D), jnp.float32),
            pltpu.VMEM((NP,), jnp.float32),
            pltpu.SemaphoreType.DMA,
            pltpu.SemaphoreType.DMA,
            pltpu.SemaphoreType.DMA,
            pltpu.SemaphoreType.DMA,
        ],
    )
    def k(x_hbm, src_hbm, dst_hbm, dstf_hbm, zsum_hbm, sum_out, cnt_out,
          accum, src_v, dst_v, dst1d, rows_v0, rows_v1, cnt_v,
          gsem0, gsem1, ssem0, ssem1):
        c = lax.axis_index("c")
        s = lax.axis_index("s")
        wid = c * NS + s
        rows_slice = pl.ds(s * ROWS_PER_TILE, ROWS_PER_TILE)
        pltpu.sync_copy(zsum_hbm.at[rows_slice], accum.at[rows_slice])
        zero16 = jnp.zeros((16,), jnp.float32)
        ones16 = jnp.ones((16,), jnp.float32)

        def zbody(i, carry):
            cnt_v[pl.ds(i * 16, 16)] = zero16
            return carry

        lax.fori_loop(0, NP // 16, zbody, 0)
        plsc.subcore_barrier()

        def outer(g, carry):
            goff = pl.multiple_of(g * GB, GB)
            goff2 = pl.multiple_of(g * GB * CHUNK, GB * CHUNK)
            pltpu.sync_copy(src_hbm.at[wid].at[pl.ds(goff, GB)], src_v)
            pltpu.sync_copy(dst_hbm.at[wid].at[pl.ds(goff, GB)], dst_v)
            pltpu.sync_copy(dstf_hbm.at[wid].at[pl.ds(goff2, GB * CHUNK)], dst1d)

            rows = (rows_v0, rows_v1)
            gsem = (gsem0, gsem1)
            ssem = (ssem0, ssem1)
            gath = [None, None]
            scat = [None, None]
            gath[0] = pltpu.async_copy(x_hbm.at[src_v.at[0]], rows[0], gsem[0])
            for j in range(GB):
                b = j % 2
                o = 1 - b
                if j + 1 < GB:
                    if scat[o] is not None:
                        scat[o].wait()
                        scat[o] = None
                    gath[o] = pltpu.async_copy(
                        x_hbm.at[src_v.at[j + 1]], rows[o], gsem[o])
                gath[b].wait()
                scat[b] = pltpu.async_copy(
                    rows[b], accum.at[dst_v.at[j]], ssem[b], add=True)

                def cbody(m, carry2, _j=j):
                    vals = dst1d[pl.ds(_j * CHUNK + m * 16, 16)]
                    plsc.addupdate_scatter(cnt_v, [vals], ones16)
                    return carry2

                lax.fori_loop(0, CHUNK // 16, cbody, 0)
            for b in (0, 1):
                if scat[b] is not None:
                    scat[b].wait()
            return carry

        lax.fori_loop(0, NCHUNKS // GB, outer, 0)
        plsc.subcore_barrier()
        pltpu.sync_copy(accum.at[rows_slice], sum_out.at[c].at[rows_slice])
        pltpu.sync_copy(cnt_v, cnt_out.at[wid])

    return k(x, src_r, dst_r, dst_f, zsum)


def _tc_combine(x, w1, w2, sp, cp, n_dummy):
    R = 1000

    def body(x_ref, w1_ref, w2_ref, sp_ref, cp_ref, o_ref):
        j = pl.program_id(0)
        ssum = sp_ref[0] + sp_ref[1]
        cnt = jnp.sum(cp_ref[...], axis=1, keepdims=True)
        rows = lax.broadcasted_iota(jnp.int32, (R, 1), 0)
        corr = jnp.where(jnp.logical_and(rows == 0, j == 0),
                         jnp.float32(n_dummy), jnp.float32(0.0))
        xb = x_ref[...]
        ssum = ssum - corr * xb
        cnt = jnp.maximum(cnt - corr, 1.0)
        mean = ssum / cnt
        o_ref[...] = (
            jnp.dot(xb, w1_ref[...], preferred_element_type=jnp.float32)
            + jnp.dot(mean, w2_ref[...], preferred_element_type=jnp.float32))

    return pl.pallas_call(
        body,
        grid=(N // R,),
        in_specs=[
            pl.BlockSpec((R, D), lambda j: (j, 0)),
            pl.BlockSpec((D, D), lambda j: (0, 0)),
            pl.BlockSpec((D, D), lambda j: (0, 0)),
            pl.BlockSpec((NC, R, D), lambda j: (0, j, 0)),
            pl.BlockSpec((R, NW), lambda j: (j, 0)),
        ],
        out_specs=pl.BlockSpec((R, D), lambda j: (j, 0)),
        out_shape=jax.ShapeDtypeStruct((N, D), jnp.float32),
    )(x, w1, w2, sp, cp)


def kernel(x, edge_index, weight):
    src = edge_index[0]
    dst = edge_index[1]
    e = src.shape[0]
    pad = E_PAD - e
    src_p = jnp.concatenate([src, jnp.zeros((pad,), jnp.int32)])
    dst_p = jnp.concatenate([dst, jnp.zeros((pad,), jnp.int32)])
    src_r = src_p.reshape(NW, NCHUNKS, CHUNK)
    dst_r = dst_p.reshape(NW, NCHUNKS, CHUNK)
    dst_f = dst_p.reshape(NW, EDGES_PER_TILE)
    zsum = jnp.zeros((NP, D), jnp.float32)
    sp, cnt_parts = _sc_aggregate(x, src_r, dst_r, dst_f, zsum)
    sp = sp[:, :N]
    cp = cnt_parts[:, :N].T
    w1 = weight[:D]
    w2 = weight[D:]
    return _tc_combine(x, w1, w2, sp, cp, float(pad))


# speedup vs baseline: 3.9014x; 1.0012x over previous
import functools

import jax
import jax.numpy as jnp
from jax import lax
from jax.experimental import pallas as pl
from jax.experimental.pallas import tpu as pltpu
from jax.experimental.pallas import tpu_sc as plsc

N = 10000
D = 128
NC = 2
NS = 16
NW = NC * NS
CHUNK = 128
NCHUNKS = 80
GB = 8
EDGES_PER_TILE = NCHUNKS * CHUNK
E_PAD = NW * EDGES_PER_TILE
ROWS_PER_TILE = 632
NP = NS * ROWS_PER_TILE


def _sc_aggregate(x, src_r, dst_r, dst_f, zsum):
    mesh = plsc.VectorSubcoreMesh(core_axis_name="c", subcore_axis_name="s")

    @functools.partial(
        pl.kernel,
        mesh=mesh,
        compiler_params=pltpu.CompilerParams(needs_layout_passes=False),
        out_type=(
            jax.ShapeDtypeStruct((NC, NP, D), jnp.float32),
            jax.ShapeDtypeStruct((NW, NP), jnp.float32),
        ),
        scratch_types=[
            pltpu.VMEM_SHARED((NP, D), jnp.float32),
            pltpu.VMEM((GB, CHUNK), jnp.int32),
            pltpu.VMEM((GB, CHUNK), jnp.int32),
            pltpu.VMEM((GB * CHUNK,), jnp.int32),
            pltpu.VMEM((CHUNK, D), jnp.float32),
            pltpu.VMEM((CHUNK, D), jnp.float32),
            pltpu.VMEM((NP,), jnp.float32),
            pltpu.SemaphoreType.DMA,
            pltpu.SemaphoreType.DMA,
            pltpu.SemaphoreType.DMA,
            pltpu.SemaphoreType.DMA,
        ],
    )
    def k(x_hbm, src_hbm, dst_hbm, dstf_hbm, zsum_hbm, sum_out, cnt_out,
          accum, src_v, dst_v, dst1d, rows_v0, rows_v1, cnt_v,
          gsem0, gsem1, ssem0, ssem1):
        c = lax.axis_index("c")
        s = lax.axis_index("s")
        wid = c * NS + s
        rows_slice = pl.ds(s * ROWS_PER_TILE, ROWS_PER_TILE)
        pltpu.sync_copy(zsum_hbm.at[rows_slice], accum.at[rows_slice])
        zero16 = jnp.zeros((16,), jnp.float32)
        ones16 = jnp.ones((16,), jnp.float32)

        def zbody(i, carry):
            cnt_v[pl.ds(i * 16, 16)] = zero16
            return carry

        lax.fori_loop(0, NP // 16, zbody, 0)
        plsc.subcore_barrier()

        def outer(g, carry):
            goff = pl.multiple_of(g * GB, GB)
            goff2 = pl.multiple_of(g * GB * CHUNK, GB * CHUNK)
            pltpu.sync_copy(src_hbm.at[wid].at[pl.ds(goff, GB)], src_v)
            pltpu.sync_copy(dst_hbm.at[wid].at[pl.ds(goff, GB)], dst_v)
            pltpu.sync_copy(dstf_hbm.at[wid].at[pl.ds(goff2, GB * CHUNK)], dst1d)

            rows = (rows_v0, rows_v1)
            gsem = (gsem0, gsem1)
            ssem = (ssem0, ssem1)
            gath = [None, None]
            scat = [None, None]
            gath[0] = pltpu.async_copy(x_hbm.at[src_v.at[0]], rows[0], gsem[0])
            for j in range(GB):
                b = j % 2
                o = 1 - b
                if j + 1 < GB:
                    if scat[o] is not None:
                        scat[o].wait()
                        scat[o] = None
                    gath[o] = pltpu.async_copy(
                        x_hbm.at[src_v.at[j + 1]], rows[o], gsem[o])
                gath[b].wait()
                scat[b] = pltpu.async_copy(
                    rows[b], accum.at[dst_v.at[j]], ssem[b], add=True)

                def cbody(m, carry2, _j=j):
                    vals = dst1d[pl.ds(_j * CHUNK + m * 16, 16)]
                    plsc.addupdate_scatter(cnt_v, [vals], ones16)
                    return carry2

                lax.fori_loop(0, CHUNK // 16, cbody, 0)
            for b in (0, 1):
                if scat[b] is not None:
                    scat[b].wait()
            return carry

        lax.fori_loop(0, NCHUNKS // GB, outer, 0)
        plsc.subcore_barrier()
        pltpu.sync_copy(accum.at[rows_slice], sum_out.at[c].at[rows_slice])
        pltpu.sync_copy(cnt_v, cnt_out.at[wid])

    return k(x, src_r, dst_r, dst_f, zsum)


def _tc_combine(x, w1, w2, sp, cp):
    R = 1000

    def body(x_ref, w1_ref, w2_ref, sp_ref, cp_ref, o_ref):
        ssum = sp_ref[0] + sp_ref[1]
        cnt = jnp.sum(cp_ref[...], axis=1, keepdims=True)
        xb = x_ref[...]
        cnt = jnp.maximum(cnt, 1.0)
        mean = ssum / cnt
        o_ref[...] = (
            jnp.dot(xb, w1_ref[...], preferred_element_type=jnp.float32)
            + jnp.dot(mean, w2_ref[...], preferred_element_type=jnp.float32))

    return pl.pallas_call(
        body,
        grid=(N // R,),
        in_specs=[
            pl.BlockSpec((R, D), lambda j: (j, 0)),
            pl.BlockSpec((D, D), lambda j: (0, 0)),
            pl.BlockSpec((D, D), lambda j: (0, 0)),
            pl.BlockSpec((NC, R, D), lambda j: (0, j, 0)),
            pl.BlockSpec((R, NW), lambda j: (j, 0)),
        ],
        out_specs=pl.BlockSpec((R, D), lambda j: (j, 0)),
        out_shape=jax.ShapeDtypeStruct((N, D), jnp.float32),
    )(x, w1, w2, sp, cp)


def kernel(x, edge_index, weight):
    src = edge_index[0]
    dst = edge_index[1]
    e = src.shape[0]
    pad = E_PAD - e
    trash = N + (jnp.arange(pad, dtype=jnp.int32) % (NP - N))
    src_p = jnp.concatenate([src, jnp.zeros((pad,), jnp.int32)])
    dst_p = jnp.concatenate([dst, trash])
    src_r = src_p.reshape(NW, NCHUNKS, CHUNK)
    dst_r = dst_p.reshape(NW, NCHUNKS, CHUNK)
    dst_f = dst_p.reshape(NW, EDGES_PER_TILE)
    zsum = jnp.zeros((NP, D), jnp.float32)
    sp, cnt_parts = _sc_aggregate(x, src_r, dst_r, dst_f, zsum)
    sp = sp[:, :N]
    cp = cnt_parts[:, :N].T
    w1 = weight[:D]
    w2 = weight[D:]
    return _tc_combine(x, w1, w2, sp, cp)


# speedup vs baseline: 7.8190x; 2.0041x over previous
import functools

import jax
import jax.numpy as jnp
from jax import lax
from jax.experimental import pallas as pl
from jax.experimental.pallas import tpu as pltpu
from jax.experimental.pallas import tpu_sc as plsc

N = 10000
D = 128
H = 64
NC = 2
NS = 16
NW = NC * NS
CHUNK = 128
NCHUNKS = 160
GB = 8
EDGES_PER_TILE = NCHUNKS * CHUNK
E_PAD = NS * EDGES_PER_TILE
ROWS_PER_TILE = 632
NP = NS * ROWS_PER_TILE


def _sc_aggregate(xs, src_r, dst_r, zsum):
    mesh = plsc.VectorSubcoreMesh(core_axis_name="c", subcore_axis_name="s")

    @functools.partial(
        pl.kernel,
        mesh=mesh,
        compiler_params=pltpu.CompilerParams(needs_layout_passes=False,
                                             use_tc_tiling_on_sc=False),
        out_type=(
            jax.ShapeDtypeStruct((NC, NP, H), jnp.float32),
            jax.ShapeDtypeStruct((NW, NP), jnp.float32),
        ),
        scratch_types=[
            pltpu.VMEM_SHARED((NP, H), jnp.float32),
            pltpu.VMEM_SHARED((NP, H), jnp.float32),
            pltpu.VMEM((GB, CHUNK), jnp.int32),
            pltpu.VMEM((GB, CHUNK), jnp.int32),
            pltpu.VMEM((CHUNK, H), jnp.float32),
            pltpu.VMEM((CHUNK, H), jnp.float32),
            pltpu.VMEM((NP,), jnp.float32),
            pltpu.SemaphoreType.DMA,
            pltpu.SemaphoreType.DMA,
            pltpu.SemaphoreType.DMA,
            pltpu.SemaphoreType.DMA,
        ],
    )
    def k(xs_hbm, src_hbm, dst_hbm, zsum_hbm, sum_out, cnt_out,
          xsp, accum, src_v, dst_v, rows_v0, rows_v1, cnt_v,
          gsem0, gsem1, ssem0, ssem1):
        c = lax.axis_index("c")
        s = lax.axis_index("s")
        wid = c * NS + s
        rows_slice = pl.ds(s * ROWS_PER_TILE, ROWS_PER_TILE)
        pltpu.sync_copy(xs_hbm.at[c].at[rows_slice], xsp.at[rows_slice])
        pltpu.sync_copy(zsum_hbm.at[rows_slice], accum.at[rows_slice])
        zero16 = jnp.zeros((16,), jnp.float32)
        ones16 = jnp.ones((16,), jnp.float32)

        def zbody(i, carry):
            cnt_v[pl.ds(i * 16, 16)] = zero16
            return carry

        lax.fori_loop(0, NP // 16, zbody, 0)
        plsc.subcore_barrier()

        def outer(g, carry):
            goff = pl.multiple_of(g * GB, GB)
            pltpu.sync_copy(src_hbm.at[s].at[pl.ds(goff, GB)], src_v)
            pltpu.sync_copy(dst_hbm.at[s].at[pl.ds(goff, GB)], dst_v)

            rows = (rows_v0, rows_v1)
            gsem = (gsem0, gsem1)
            ssem = (ssem0, ssem1)
            gath = [None, None]
            scat = [None, None]
            gath[0] = pltpu.async_copy(xsp.at[src_v.at[0]], rows[0], gsem[0])
            for j in range(GB):
                b = j % 2
                o = 1 - b
                if j + 1 < GB:
                    if scat[o] is not None:
                        scat[o].wait()
                        scat[o] = None
                    gath[o] = pltpu.async_copy(
                        xsp.at[src_v.at[j + 1]], rows[o], gsem[o])
                gath[b].wait()
                scat[b] = pltpu.async_copy(
                    rows[b], accum.at[dst_v.at[j]], ssem[b], add=True)

                def cbody(m, carry2, _j=j):
                    vals = dst_v[_j, pl.ds(m * 16, 16)]
                    plsc.addupdate_scatter(cnt_v, [vals], ones16)
                    return carry2

                lax.fori_loop(0, CHUNK // 16, cbody, 0)
            for b in (0, 1):
                if scat[b] is not None:
                    scat[b].wait()
            return carry

        lax.fori_loop(0, NCHUNKS // GB, outer, 0)
        plsc.subcore_barrier()
        pltpu.sync_copy(accum.at[rows_slice], sum_out.at[c].at[rows_slice])
        pltpu.sync_copy(cnt_v, cnt_out.at[wid])

    return k(xs, src_r, dst_r, zsum)


def _tc_combine(x, w1, w2, sp, cp):
    R = 1000

    def body(x_ref, w1_ref, w2_ref, sp_ref, cp_ref, o_ref):
        ssum = sp_ref[...]
        cnt = 0.5 * jnp.sum(cp_ref[...], axis=1, keepdims=True)
        xb = x_ref[...]
        cnt = jnp.maximum(cnt, 1.0)
        mean = ssum / cnt
        o_ref[...] = (
            jnp.dot(xb, w1_ref[...], preferred_element_type=jnp.float32)
            + jnp.dot(mean, w2_ref[...], preferred_element_type=jnp.float32))

    return pl.pallas_call(
        body,
        grid=(N // R,),
        in_specs=[
            pl.BlockSpec((R, D), lambda j: (j, 0)),
            pl.BlockSpec((D, D), lambda j: (0, 0)),
            pl.BlockSpec((D, D), lambda j: (0, 0)),
            pl.BlockSpec((R, D), lambda j: (j, 0)),
            pl.BlockSpec((R, NW), lambda j: (j, 0)),
        ],
        out_specs=pl.BlockSpec((R, D), lambda j: (j, 0)),
        out_shape=jax.ShapeDtypeStruct((N, D), jnp.float32),
    )(x, w1, w2, sp, cp)


def kernel(x, edge_index, weight):
    src = edge_index[0]
    dst = edge_index[1]
    e = src.shape[0]
    pad = E_PAD - e
    trash = N + (jnp.arange(pad, dtype=jnp.int32) % (NP - N))
    src_p = jnp.concatenate([src, jnp.zeros((pad,), jnp.int32)])
    dst_p = jnp.concatenate([dst, trash])
    src_r = src_p.reshape(NS, NCHUNKS, CHUNK)
    dst_r = dst_p.reshape(NS, NCHUNKS, CHUNK)
    xs = jnp.pad(x, ((0, NP - N), (0, 0))).reshape(NP, NC, H).transpose(1, 0, 2)
    zsum = jnp.zeros((NP, H), jnp.float32)
    sp_halves, cnt_parts = _sc_aggregate(xs, src_r, dst_r, zsum)
    sp = jnp.concatenate([sp_halves[0, :N], sp_halves[1, :N]], axis=-1)
    cp = cnt_parts[:, :N].T
    w1 = weight[:D]
    w2 = weight[D:]
    return _tc_combine(x, w1, w2, sp, cp)


# speedup vs baseline: 8.2860x; 1.0597x over previous
import functools

import jax
import jax.numpy as jnp
from jax import lax
from jax.experimental import pallas as pl
from jax.experimental.pallas import tpu as pltpu
from jax.experimental.pallas import tpu_sc as plsc

N = 10000
D = 128
H = 64
NC = 2
NS = 16
NW = NC * NS
CHUNK = 128
NCHUNKS = 160
GB = 8
EDGES_PER_TILE = NCHUNKS * CHUNK
E_PAD = NS * EDGES_PER_TILE
ROWS_PER_TILE = 632
NP = NS * ROWS_PER_TILE


def _sc_aggregate(xs, src_r, dst_r, zsum):
    mesh = plsc.VectorSubcoreMesh(core_axis_name="c", subcore_axis_name="s")

    @functools.partial(
        pl.kernel,
        mesh=mesh,
        compiler_params=pltpu.CompilerParams(needs_layout_passes=False,
                                             use_tc_tiling_on_sc=False),
        out_type=(
            jax.ShapeDtypeStruct((NC, NP, H), jnp.float32),
            jax.ShapeDtypeStruct((NW, NP), jnp.float32),
        ),
        scratch_types=[
            pltpu.VMEM_SHARED((NP, H), jnp.float32),
            pltpu.VMEM_SHARED((NP, H), jnp.float32),
            pltpu.VMEM((GB, CHUNK), jnp.int32),
            pltpu.VMEM((GB, CHUNK), jnp.int32),
            pltpu.VMEM((CHUNK, H), jnp.float32),
            pltpu.VMEM((CHUNK, H), jnp.float32),
            pltpu.VMEM((CHUNK, H), jnp.float32),
            pltpu.VMEM((CHUNK, H), jnp.float32),
            pltpu.VMEM((NP,), jnp.float32),
            pltpu.SemaphoreType.DMA,
            pltpu.SemaphoreType.DMA,
            pltpu.SemaphoreType.DMA,
            pltpu.SemaphoreType.DMA,
            pltpu.SemaphoreType.DMA,
            pltpu.SemaphoreType.DMA,
            pltpu.SemaphoreType.DMA,
            pltpu.SemaphoreType.DMA,
        ],
    )
    def k(xs_hbm, src_hbm, dst_hbm, zsum_hbm, sum_out, cnt_out,
          xsp, accum, src_v, dst_v, rows_v0, rows_v1, rows_v2, rows_v3, cnt_v,
          gsem0, gsem1, gsem2, gsem3, ssem0, ssem1, ssem2, ssem3):
        c = lax.axis_index("c")
        s = lax.axis_index("s")
        wid = c * NS + s
        rows_slice = pl.ds(s * ROWS_PER_TILE, ROWS_PER_TILE)
        pltpu.sync_copy(xs_hbm.at[c].at[rows_slice], xsp.at[rows_slice])
        pltpu.sync_copy(zsum_hbm.at[rows_slice], accum.at[rows_slice])
        zero16 = jnp.zeros((16,), jnp.float32)
        ones16 = jnp.ones((16,), jnp.float32)

        def zbody(i, carry):
            cnt_v[pl.ds(i * 16, 16)] = zero16
            return carry

        lax.fori_loop(0, NP // 16, zbody, 0)
        plsc.subcore_barrier()

        def outer(g, carry):
            goff = pl.multiple_of(g * GB, GB)
            pltpu.sync_copy(src_hbm.at[s].at[pl.ds(goff, GB)], src_v)
            pltpu.sync_copy(dst_hbm.at[s].at[pl.ds(goff, GB)], dst_v)

            NBUF = 4
            rows = (rows_v0, rows_v1, rows_v2, rows_v3)
            gsem = (gsem0, gsem1, gsem2, gsem3)
            ssem = (ssem0, ssem1, ssem2, ssem3)
            gath = [None] * NBUF
            scat = [None] * NBUF
            for j in range(NBUF - 1):
                gath[j] = pltpu.async_copy(
                    xsp.at[src_v.at[j]], rows[j], gsem[j])
            for j in range(GB):
                b = j % NBUF
                if j + NBUF - 1 < GB:
                    o = (j + NBUF - 1) % NBUF
                    if scat[o] is not None:
                        scat[o].wait()
                        scat[o] = None
                    gath[o] = pltpu.async_copy(
                        xsp.at[src_v.at[j + NBUF - 1]], rows[o], gsem[o])
                gath[b].wait()
                scat[b] = pltpu.async_copy(
                    rows[b], accum.at[dst_v.at[j]], ssem[b], add=True)

                def cbody(m, carry2, _j=j):
                    vals = dst_v[_j, pl.ds(m * 16, 16)]
                    plsc.addupdate_scatter(cnt_v, [vals], ones16)
                    return carry2

                lax.fori_loop(0, CHUNK // 16, cbody, 0)
            for b in range(NBUF):
                if scat[b] is not None:
                    scat[b].wait()
            return carry

        lax.fori_loop(0, NCHUNKS // GB, outer, 0)
        plsc.subcore_barrier()
        pltpu.sync_copy(accum.at[rows_slice], sum_out.at[c].at[rows_slice])
        pltpu.sync_copy(cnt_v, cnt_out.at[wid])

    return k(xs, src_r, dst_r, zsum)


def _tc_combine(x, w1, w2, sp, cp):
    R = 1000

    def body(x_ref, w1_ref, w2_ref, sp_ref, cp_ref, o_ref):
        ssum = sp_ref[...]
        cnt = 0.5 * jnp.sum(cp_ref[...], axis=1, keepdims=True)
        xb = x_ref[...]
        cnt = jnp.maximum(cnt, 1.0)
        mean = ssum / cnt
        o_ref[...] = (
            jnp.dot(xb, w1_ref[...], preferred_element_type=jnp.float32)
            + jnp.dot(mean, w2_ref[...], preferred_element_type=jnp.float32))

    return pl.pallas_call(
        body,
        grid=(N // R,),
        in_specs=[
            pl.BlockSpec((R, D), lambda j: (j, 0)),
            pl.BlockSpec((D, D), lambda j: (0, 0)),
            pl.BlockSpec((D, D), lambda j: (0, 0)),
            pl.BlockSpec((R, D), lambda j: (j, 0)),
            pl.BlockSpec((R, NW), lambda j: (j, 0)),
        ],
        out_specs=pl.BlockSpec((R, D), lambda j: (j, 0)),
        out_shape=jax.ShapeDtypeStruct((N, D), jnp.float32),
    )(x, w1, w2, sp, cp)


def kernel(x, edge_index, weight):
    src = edge_index[0]
    dst = edge_index[1]
    e = src.shape[0]
    pad = E_PAD - e
    trash = N + (jnp.arange(pad, dtype=jnp.int32) % (NP - N))
    src_p = jnp.concatenate([src, jnp.zeros((pad,), jnp.int32)])
    dst_p = jnp.concatenate([dst, trash])
    src_r = src_p.reshape(NS, NCHUNKS, CHUNK)
    dst_r = dst_p.reshape(NS, NCHUNKS, CHUNK)
    xs = jnp.pad(x, ((0, NP - N), (0, 0))).reshape(NP, NC, H).transpose(1, 0, 2)
    zsum = jnp.zeros((NP, H), jnp.float32)
    sp_halves, cnt_parts = _sc_aggregate(xs, src_r, dst_r, zsum)
    sp = jnp.concatenate([sp_halves[0, :N], sp_halves[1, :N]], axis=-1)
    cp = cnt_parts[:, :N].T
    w1 = weight[:D]
    w2 = weight[D:]
    return _tc_combine(x, w1, w2, sp, cp)
